# baseline (speedup 1.0000x reference)
import functools

import jax
import jax.numpy as jnp
from jax import lax
from jax.experimental import pallas as pl
from jax.experimental.pallas import tpu as pltpu
from jax.experimental.pallas import tpu_sc as plsc

BS, N, LIE, DV, DE = 4, 1024, 6, 512, 4
M = 256
ALPHA = 0.2
SB = 256
NS = N // SB


def _fps_body(a_ref, x0, x1, x2, x3, x4, x5, chosen_ref, scratch):
    b = pl.program_id(0)
    s = pl.program_id(1)
    r = jnp.sqrt(x0[0] * x0[0] + x1[0] * x1[0] + x2[0] * x2[0])
    t = jnp.sqrt(x3[0] * x3[0] + x4[0] * x4[0] + x5[0] * x5[0])
    y = ALPHA * r + (1.0 - ALPHA) * t
    scratch[pl.ds(s * SB * 8, SB * 8), :] = y.reshape(SB * 8, 128)

    @pl.when(s == NS - 1)
    def _():
        riota = jax.lax.broadcasted_iota(jnp.int32, (8, 128), 0)
        liota = jax.lax.broadcasted_iota(jnp.int32, (8, 128), 1)
        flat = riota * 128 + liota

        def body(i, carry):
            d, f = carry
            chosen_ref[b, i] = f
            row = scratch[pl.ds(f * 8, 8), :]
            d2 = jnp.minimum(d, row)
            gmax = jnp.max(d2)
            cand = jnp.where(d2 == gmax, flat, jnp.int32(2**30))
            f2 = jnp.min(cand)
            return d2, f2

        d0 = jnp.full((8, 128), 1e8, jnp.float32)
        jax.lax.fori_loop(0, M, body, (d0, a_ref[b]))


def _fps_chosen(planes, interpret=False):
    a = jax.random.randint(jax.random.key(1), (BS,), 0, N).astype(jnp.int32)
    bspec = pl.BlockSpec((1, SB, N), lambda b, s: (b, s, 0))
    return pl.pallas_call(
        _fps_body,
        grid=(BS, NS),
        in_specs=[pl.BlockSpec(memory_space=pltpu.SMEM)] + [bspec] * LIE,
        out_specs=pl.BlockSpec(memory_space=pltpu.SMEM),
        out_shape=jax.ShapeDtypeStruct((BS, M), jnp.int32),
        scratch_shapes=[pltpu.VMEM((N * 8, 128), jnp.float32)],
        interpret=interpret,
        compiler_params=pltpu.CompilerParams(
            dimension_semantics=("arbitrary", "arbitrary")),
    )(a, *planes)


NW = 32
RPW = BS * M // NW


def _sc_gather_rows(tbl, idxflat):
    d = tbl.shape[1]
    mesh = plsc.VectorSubcoreMesh(core_axis_name="c", subcore_axis_name="s")

    @functools.partial(
        pl.kernel, mesh=mesh,
        out_type=jax.ShapeDtypeStruct((BS * M, d), jnp.float32),
        scratch_types=[
            pltpu.VMEM((RPW,), jnp.int32),
            pltpu.VMEM((RPW, d), jnp.float32),
            pltpu.SemaphoreType.DMA,
        ],
    )
    def k(tbl_hbm, idx_hbm, out_hbm, idx_v, rows_v, sem):
        wid = lax.axis_index("s") * 2 + lax.axis_index("c")
        base = wid * RPW
        pltpu.sync_copy(idx_hbm.at[pl.ds(base, RPW)], idx_v)
        pltpu.async_copy(tbl_hbm.at[idx_v], rows_v, sem).wait()
        pltpu.sync_copy(rows_v, out_hbm.at[pl.ds(base, RPW)])

    return k(tbl, idxflat)


def _sc_rowgather_multi(tbls, ids, d):
    n = len(tbls)
    mesh = plsc.VectorSubcoreMesh(core_axis_name="c", subcore_axis_name="s")

    @functools.partial(
        pl.kernel, mesh=mesh,
        out_type=[jax.ShapeDtypeStruct((BS * M, d), jnp.float32)] * n,
        scratch_types=[
            pltpu.VMEM((RPW,), jnp.int32),
            pltpu.VMEM((RPW, d), jnp.float32),
            pltpu.VMEM((RPW, d), jnp.float32),
            pltpu.SemaphoreType.DMA,
            pltpu.SemaphoreType.DMA,
        ],
    )
    def k(*refs):
        tbl_refs = refs[:n]
        ids_hbm = refs[n]
        outs = refs[n + 1:2 * n + 1]
        idx_v = refs[2 * n + 1]
        rows = [refs[2 * n + 2], refs[2 * n + 3]]
        sems = [refs[2 * n + 4], refs[2 * n + 5]]
        wid = lax.axis_index("s") * 2 + lax.axis_index("c")
        base = wid * RPW
        pltpu.sync_copy(ids_hbm.at[pl.ds(base, RPW)], idx_v)
        hs = [None, None]
        hs[0] = pltpu.async_copy(tbl_refs[0].at[idx_v], rows[0], sems[0])
        for c in range(n):
            if c + 1 < n:
                hs[(c + 1) % 2] = pltpu.async_copy(
                    tbl_refs[c + 1].at[idx_v], rows[(c + 1) % 2], sems[(c + 1) % 2])
            hs[c % 2].wait()
            pltpu.sync_copy(rows[c % 2], outs[c].at[pl.ds(base, RPW)])

    return k(*tbls, ids)


def _transpose_body(*refs):
    n = len(refs) // 2
    for c in range(n):
        refs[n + c][0] = refs[c][0].T


def _tc_transpose(arrs):
    n = len(arrs)
    TB = 128
    return pl.pallas_call(
        _transpose_body,
        grid=(BS, N // TB),
        in_specs=[pl.BlockSpec((1, M, TB), lambda b, t: (b, 0, t))] * n,
        out_specs=[pl.BlockSpec((1, TB, M), lambda b, t: (b, t, 0))] * n,
        out_shape=[jax.ShapeDtypeStruct((BS, N, M), jnp.float32)] * n,
        compiler_params=pltpu.CompilerParams(
            dimension_semantics=("arbitrary", "arbitrary")),
    )(*arrs)


def _repack_body(*refs):
    out_a, out_e = refs[-2], refs[-1]
    for c in range(LIE):
        out_a[0, :, :, c] = refs[c][0]
    for c in range(DE):
        out_e[0, :, :, c] = refs[LIE + c][0]


def _repack(chplanes):
    RB = 16
    return pl.pallas_call(
        _repack_body,
        grid=(BS, M // RB),
        in_specs=[pl.BlockSpec((1, RB, M), lambda b, s: (b, s, 0))] * (LIE + DE),
        out_specs=[
            pl.BlockSpec((1, RB, M, LIE), lambda b, s: (b, s, 0, 0)),
            pl.BlockSpec((1, RB, M, DE), lambda b, s: (b, s, 0, 0)),
        ],
        out_shape=[
            jax.ShapeDtypeStruct((BS, M, M, LIE), jnp.float32),
            jax.ShapeDtypeStruct((BS, M, M, DE), jnp.float32),
        ],
        compiler_params=pltpu.CompilerParams(
            dimension_semantics=("arbitrary", "arbitrary")),
    )(*chplanes)


def kernel(abq_pairs, vals, mask, edges):
    planes = [abq_pairs[..., c] for c in range(LIE)]
    eplanes = [edges[..., c] for c in range(DE)]
    qidx = _fps_chosen(planes)
    idxflat = (qidx + jnp.arange(BS, dtype=jnp.int32)[:, None] * N).reshape(BS * M)
    allp = [p.reshape(BS * N, N) for p in planes + eplanes]
    tmp = _sc_rowgather_multi(allp, idxflat, N)
    tmpT = _tc_transpose([t.reshape(BS, M, N) for t in tmp])
    g = _sc_rowgather_multi([t.reshape(BS * N, M) for t in tmpT], idxflat, M)
    sub_abq, sub_edges = _repack([t.reshape(BS, M, M) for t in g])
    sub_vals = _sc_gather_rows(vals.reshape(BS * N, DV), idxflat).reshape(BS, M, DV)
    sub_mask = jnp.ones((BS, M), dtype=mask.dtype)
    return sub_abq, sub_vals, sub_mask, sub_edges


# speedup vs baseline: 1.8457x; 1.8457x over previous
import functools

import jax
import jax.numpy as jnp
from jax import lax
from jax.experimental import pallas as pl
from jax.experimental.pallas import tpu as pltpu
from jax.experimental.pallas import tpu_sc as plsc

BS, N, LIE, DV, DE = 4, 1024, 6, 512, 4
M = 256
ALPHA = 0.2
SB = 256
NS = N // SB


def _fps_body(a_ref, x0, x1, x2, x3, x4, x5, chosen_ref, scratch):
    b = pl.program_id(0)
    s = pl.program_id(1)
    r = jnp.sqrt(x0[0] * x0[0] + x1[0] * x1[0] + x2[0] * x2[0])
    t = jnp.sqrt(x3[0] * x3[0] + x4[0] * x4[0] + x5[0] * x5[0])
    y = ALPHA * r + (1.0 - ALPHA) * t
    scratch[pl.ds(s * SB * 8, SB * 8), :] = y.reshape(SB * 8, 128)

    @pl.when(s == NS - 1)
    def _():
        riota = jax.lax.broadcasted_iota(jnp.int32, (8, 128), 0)
        liota = jax.lax.broadcasted_iota(jnp.int32, (8, 128), 1)
        flat = riota * 128 + liota

        def body(i, carry):
            d, f = carry
            chosen_ref[b, i] = f
            row = scratch[pl.ds(f * 8, 8), :]
            d2 = jnp.minimum(d, row)
            gmax = jnp.max(d2)
            cand = jnp.where(d2 == gmax, flat, jnp.int32(2**30))
            f2 = jnp.min(cand)
            return d2, f2

        d0 = jnp.full((8, 128), 1e8, jnp.float32)
        jax.lax.fori_loop(0, M, body, (d0, a_ref[b]))


def _fps_chosen(planes, interpret=False):
    a = jax.random.randint(jax.random.key(1), (BS,), 0, N).astype(jnp.int32)
    bspec = pl.BlockSpec((1, SB, N), lambda b, s: (b, s, 0))
    return pl.pallas_call(
        _fps_body,
        grid=(BS, NS),
        in_specs=[pl.BlockSpec(memory_space=pltpu.SMEM)] + [bspec] * LIE,
        out_specs=pl.BlockSpec(memory_space=pltpu.SMEM),
        out_shape=jax.ShapeDtypeStruct((BS, M), jnp.int32),
        scratch_shapes=[pltpu.VMEM((N * 8, 128), jnp.float32)],
        interpret=interpret,
        compiler_params=pltpu.CompilerParams(
            dimension_semantics=("parallel", "arbitrary")),
    )(a, *planes)


NW = 32
RPW = BS * M // NW


def _sc_gather_rows(tbl, idxflat):
    d = tbl.shape[1]
    mesh = plsc.VectorSubcoreMesh(core_axis_name="c", subcore_axis_name="s")

    @functools.partial(
        pl.kernel, mesh=mesh,
        out_type=jax.ShapeDtypeStruct((BS * M, d), jnp.float32),
        scratch_types=[
            pltpu.VMEM((RPW,), jnp.int32),
            pltpu.VMEM((RPW, d), jnp.float32),
            pltpu.SemaphoreType.DMA,
        ],
    )
    def k(tbl_hbm, idx_hbm, out_hbm, idx_v, rows_v, sem):
        wid = lax.axis_index("s") * 2 + lax.axis_index("c")
        base = wid * RPW
        pltpu.sync_copy(idx_hbm.at[pl.ds(base, RPW)], idx_v)
        pltpu.async_copy(tbl_hbm.at[idx_v], rows_v, sem).wait()
        pltpu.sync_copy(rows_v, out_hbm.at[pl.ds(base, RPW)])

    return k(tbl, idxflat)


def kernel(abq_pairs, vals, mask, edges):
    planes = [abq_pairs[..., c] for c in range(LIE)]
    qidx = _fps_chosen(planes)
    B = jnp.arange(BS)[:, None]
    idxflat = (qidx + jnp.arange(BS, dtype=jnp.int32)[:, None] * N).reshape(BS * M)
    sub_abq = abq_pairs[B, qidx][B, :, qidx]
    sub_vals = _sc_gather_rows(vals.reshape(BS * N, DV), idxflat).reshape(BS, M, DV)
    sub_mask = jnp.ones((BS, M), dtype=mask.dtype)
    sub_edges = edges[B, qidx][B, :, qidx]
    return sub_abq, sub_vals, sub_mask, sub_edges


# speedup vs baseline: 2.0949x; 1.1350x over previous
import functools

import jax
import jax.numpy as jnp
from jax import lax
from jax.experimental import pallas as pl
from jax.experimental.pallas import tpu as pltpu
from jax.experimental.pallas import tpu_sc as plsc

BS, N, LIE, DV, DE = 4, 1024, 6, 512, 4
M = 256
ALPHA = 0.2
SB = 256
NS = N // SB


def _fps_body(a_ref, x0, x1, x2, x3, x4, x5, chosen_ref, scratch):
    b = pl.program_id(0)
    s = pl.program_id(1)
    r = jnp.sqrt(x0[0] * x0[0] + x1[0] * x1[0] + x2[0] * x2[0])
    t = jnp.sqrt(x3[0] * x3[0] + x4[0] * x4[0] + x5[0] * x5[0])
    y = ALPHA * r + (1.0 - ALPHA) * t
    scratch[pl.ds((b * N + s * SB) * 8, SB * 8), :] = y.reshape(SB * 8, 128)

    @pl.when((b == BS - 1) & (s == NS - 1))
    def _():
        riota = jax.lax.broadcasted_iota(jnp.int32, (8, 128), 0)
        liota = jax.lax.broadcasted_iota(jnp.int32, (8, 128), 1)
        flat = riota * 128 + liota
        big = jnp.int32(2**30)

        def body(i, carry):
            ds_, fs = carry
            nds, nfs = [], []
            for b2 in range(BS):
                chosen_ref[b2, i] = fs[b2]
                row = scratch[pl.ds((b2 * N + fs[b2]) * 8, 8), :]
                d2 = jnp.minimum(ds_[b2], row)
                gmax = jnp.max(d2)
                cand = jnp.where(d2 == gmax, flat, big)
                nds.append(d2)
                nfs.append(jnp.min(cand))
            return tuple(nds), tuple(nfs)

        d0 = jnp.full((8, 128), 1e8, jnp.float32)
        jax.lax.fori_loop(
            0, M, body,
            ((d0,) * BS, tuple(a_ref[b2] for b2 in range(BS))))


def _fps_chosen(planes, interpret=False):
    a = jax.random.randint(jax.random.key(1), (BS,), 0, N).astype(jnp.int32)
    bspec = pl.BlockSpec((1, SB, N), lambda b, s: (b, s, 0))
    return pl.pallas_call(
        _fps_body,
        grid=(BS, NS),
        in_specs=[pl.BlockSpec(memory_space=pltpu.SMEM)] + [bspec] * LIE,
        out_specs=pl.BlockSpec(memory_space=pltpu.SMEM),
        out_shape=jax.ShapeDtypeStruct((BS, M), jnp.int32),
        scratch_shapes=[pltpu.VMEM((BS * N * 8, 128), jnp.float32)],
        interpret=interpret,
        compiler_params=pltpu.CompilerParams(
            dimension_semantics=("arbitrary", "arbitrary")),
    )(a, *planes)


NW = 32
RPW = BS * M // NW


def _sc_gather_rows(tbl, idxflat):
    d = tbl.shape[1]
    mesh = plsc.VectorSubcoreMesh(core_axis_name="c", subcore_axis_name="s")

    @functools.partial(
        pl.kernel, mesh=mesh,
        out_type=jax.ShapeDtypeStruct((BS * M, d), jnp.float32),
        scratch_types=[
            pltpu.VMEM((RPW,), jnp.int32),
            pltpu.VMEM((RPW, d), jnp.float32),
            pltpu.SemaphoreType.DMA,
        ],
    )
    def k(tbl_hbm, idx_hbm, out_hbm, idx_v, rows_v, sem):
        wid = lax.axis_index("s") * 2 + lax.axis_index("c")
        base = wid * RPW
        pltpu.sync_copy(idx_hbm.at[pl.ds(base, RPW)], idx_v)
        pltpu.async_copy(tbl_hbm.at[idx_v], rows_v, sem).wait()
        pltpu.sync_copy(rows_v, out_hbm.at[pl.ds(base, RPW)])

    return k(tbl, idxflat)


def kernel(abq_pairs, vals, mask, edges):
    planes = [abq_pairs[..., c] for c in range(LIE)]
    qidx = _fps_chosen(planes)
    B = jnp.arange(BS)[:, None]
    idxflat = (qidx + jnp.arange(BS, dtype=jnp.int32)[:, None] * N).reshape(BS * M)
    sub_abq = abq_pairs[B, qidx][B, :, qidx]
    sub_vals = _sc_gather_rows(vals.reshape(BS * N, DV), idxflat).reshape(BS, M, DV)
    sub_mask = jnp.ones((BS, M), dtype=mask.dtype)
    sub_edges = edges[B, qidx][B, :, qidx]
    return sub_abq, sub_vals, sub_mask, sub_edges


# speedup vs baseline: 2.1024x; 1.0036x over previous
import functools

import jax
import jax.numpy as jnp
from jax import lax
from jax.experimental import pallas as pl
from jax.experimental.pallas import tpu as pltpu
from jax.experimental.pallas import tpu_sc as plsc

BS, N, LIE, DV, DE = 4, 1024, 6, 512, 4
M = 256
ALPHA = 0.2
SB = 512
NS = N // SB


def _fps_body(a_ref, x0, x1, x2, x3, x4, x5, chosen_ref, scratch):
    b = pl.program_id(0)
    s = pl.program_id(1)
    r = jnp.sqrt(x0[0] * x0[0] + x1[0] * x1[0] + x2[0] * x2[0])
    t = jnp.sqrt(x3[0] * x3[0] + x4[0] * x4[0] + x5[0] * x5[0])
    y = ALPHA * r + (1.0 - ALPHA) * t
    scratch[pl.ds((b * N + s * SB) * 8, SB * 8), :] = y.reshape(SB * 8, 128)

    @pl.when((b == BS - 1) & (s == NS - 1))
    def _():
        riota = jax.lax.broadcasted_iota(jnp.int32, (8, 128), 0)
        liota = jax.lax.broadcasted_iota(jnp.int32, (8, 128), 1)
        flat = riota * 128 + liota
        big = jnp.int32(2**30)

        def body(i, carry):
            ds_, fs = carry
            nds, nfs = [], []
            for b2 in range(BS):
                chosen_ref[b2, i] = fs[b2]
                row = scratch[pl.ds((b2 * N + fs[b2]) * 8, 8), :]
                d2 = jnp.minimum(ds_[b2], row)
                gmax = jnp.max(d2)
                cand = jnp.where(d2 == gmax, flat, big)
                nds.append(d2)
                nfs.append(jnp.min(cand))
            return tuple(nds), tuple(nfs)

        d0 = jnp.full((8, 128), 1e8, jnp.float32)
        jax.lax.fori_loop(
            0, M, body,
            ((d0,) * BS, tuple(a_ref[b2] for b2 in range(BS))))


def _fps_chosen(planes, interpret=False):
    a = jax.random.randint(jax.random.key(1), (BS,), 0, N).astype(jnp.int32)
    bspec = pl.BlockSpec((1, SB, N), lambda b, s: (b, s, 0))
    return pl.pallas_call(
        _fps_body,
        grid=(BS, NS),
        in_specs=[pl.BlockSpec(memory_space=pltpu.SMEM)] + [bspec] * LIE,
        out_specs=pl.BlockSpec(memory_space=pltpu.SMEM),
        out_shape=jax.ShapeDtypeStruct((BS, M), jnp.int32),
        scratch_shapes=[pltpu.VMEM((BS * N * 8, 128), jnp.float32)],
        interpret=interpret,
        compiler_params=pltpu.CompilerParams(
            dimension_semantics=("arbitrary", "arbitrary")),
    )(a, *planes)


NW = 32
RPW = BS * M // NW


def _sc_gather_rows(tbl, idxflat):
    d = tbl.shape[1]
    mesh = plsc.VectorSubcoreMesh(core_axis_name="c", subcore_axis_name="s")

    @functools.partial(
        pl.kernel, mesh=mesh,
        out_type=jax.ShapeDtypeStruct((BS * M, d), jnp.float32),
        scratch_types=[
            pltpu.VMEM((RPW,), jnp.int32),
            pltpu.VMEM((RPW, d), jnp.float32),
            pltpu.SemaphoreType.DMA,
        ],
    )
    def k(tbl_hbm, idx_hbm, out_hbm, idx_v, rows_v, sem):
        wid = lax.axis_index("s") * 2 + lax.axis_index("c")
        base = wid * RPW
        pltpu.sync_copy(idx_hbm.at[pl.ds(base, RPW)], idx_v)
        pltpu.async_copy(tbl_hbm.at[idx_v], rows_v, sem).wait()
        pltpu.sync_copy(rows_v, out_hbm.at[pl.ds(base, RPW)])

    return k(tbl, idxflat)


def kernel(abq_pairs, vals, mask, edges):
    planes = [abq_pairs[..., c] for c in range(LIE)]
    qidx = _fps_chosen(planes)
    B = jnp.arange(BS)[:, None]
    idxflat = (qidx + jnp.arange(BS, dtype=jnp.int32)[:, None] * N).reshape(BS * M)
    sub_abq = abq_pairs[B, qidx][B, :, qidx]
    sub_vals = _sc_gather_rows(vals.reshape(BS * N, DV), idxflat).reshape(BS, M, DV)
    sub_mask = jnp.ones((BS, M), dtype=mask.dtype)
    sub_edges = edges[B, qidx][B, :, qidx]
    return sub_abq, sub_vals, sub_mask, sub_edges
